# baseline (speedup 1.0000x reference)
import jax
import jax.numpy as jnp
from jax import lax
from jax.experimental import pallas as pl
from jax.experimental.pallas import tpu as pltpu
from jax.experimental.pallas import tpu_sc as plsc

TOP_K = 2
N_EXPERTS = 8
D_MODEL = 1024
NSTREAMS = 2
SUB_BLOCK = 1024
TOKENS_PER_BLOCK = NSTREAMS * SUB_BLOCK

_SC_INFO = plsc.get_sparse_core_info()
_NC = _SC_INFO.num_cores
_NS = _SC_INFO.num_subcores
_NW = _NC * _NS
_LANES = _SC_INFO.num_lanes


def _logits_kernel(*refs):
    x_refs = refs[:NSTREAMS]
    w_ref, logits_ref = refs[NSTREAMS:]
    w = w_ref[...]
    dn = (((1,), (1,)), ((), ()))
    logits_ref[...] = jnp.concatenate(
        [jax.lax.dot_general(w, x_ref[...], dn,
                             preferred_element_type=jnp.float32)
         for x_ref in x_refs], axis=1)


def _sc_router(logits_hbm, idx_hbm, wgt_hbm, l_vmem, idx_vmem, wgt_vmem):
    wid = lax.axis_index("s") * _NC + lax.axis_index("c")
    n = l_vmem.shape[1]
    base = wid * n
    pltpu.sync_copy(logits_hbm.at[:, pl.ds(base, n)], l_vmem)

    def body(c, carry):
        off = c * _LANES
        v1 = l_vmem[0, pl.ds(off, _LANES)]
        i1 = jnp.zeros((_LANES,), jnp.int32)
        v2 = jnp.full((_LANES,), -jnp.inf, jnp.float32)
        i2 = jnp.zeros((_LANES,), jnp.int32)
        for e in range(1, N_EXPERTS):
            v = l_vmem[e, pl.ds(off, _LANES)]
            ev = jnp.full((_LANES,), e, jnp.int32)
            new_top = v > v1
            beats_2nd = v > v2
            v2 = jnp.where(new_top, v1, jnp.where(beats_2nd, v, v2))
            i2 = jnp.where(new_top, i1, jnp.where(beats_2nd, ev, i2))
            v1 = jnp.where(new_top, v, v1)
            i1 = jnp.where(new_top, ev, i1)
        d = jnp.exp(v2 - v1)
        w1 = 1.0 / (1.0 + d)
        idx_vmem[0, pl.ds(off, _LANES)] = i1
        idx_vmem[1, pl.ds(off, _LANES)] = i2
        wgt_vmem[0, pl.ds(off, _LANES)] = w1
        wgt_vmem[1, pl.ds(off, _LANES)] = d * w1
        return carry

    lax.fori_loop(0, n // _LANES, body, 0)
    pltpu.sync_copy(idx_vmem, idx_hbm.at[:, pl.ds(base, n)])
    pltpu.sync_copy(wgt_vmem, wgt_hbm.at[:, pl.ds(base, n)])


@jax.jit
def kernel(hidden_states, weight):
    h = hidden_states.shape[-1]
    x = hidden_states.reshape(-1, h).astype(jnp.float32)
    t = x.shape[0]
    w = weight.astype(jnp.float32)
    b = TOKENS_PER_BLOCK
    grid = (t // b,)

    def make_spec(j):
        return pl.BlockSpec((SUB_BLOCK, h), lambda i, j=j: (i * NSTREAMS + j, 0))

    logits_t = pl.pallas_call(
        _logits_kernel,
        grid=grid,
        in_specs=[make_spec(j) for j in range(NSTREAMS)] + [
            pl.BlockSpec((N_EXPERTS, h), lambda i: (0, 0)),
        ],
        out_specs=pl.BlockSpec((N_EXPERTS, b), lambda i: (0, i)),
        out_shape=jax.ShapeDtypeStruct((N_EXPERTS, t), jnp.float32),
    )(*([x] * NSTREAMS), w)

    n_per_tec = t // _NW
    mesh = plsc.VectorSubcoreMesh(core_axis_name="c", subcore_axis_name="s")
    idx_t, wgt_t = pl.kernel(
        _sc_router,
        out_type=[
            jax.ShapeDtypeStruct((TOP_K, t), jnp.int32),
            jax.ShapeDtypeStruct((TOP_K, t), jnp.float32),
        ],
        mesh=mesh,
        scratch_types=[
            pltpu.VMEM((N_EXPERTS, n_per_tec), jnp.float32),
            pltpu.VMEM((TOP_K, n_per_tec), jnp.int32),
            pltpu.VMEM((TOP_K, n_per_tec), jnp.float32),
        ],
    )(logits_t)
    return (idx_t.T, wgt_t.T)


# speedup vs baseline: 1.4555x; 1.4555x over previous
import jax
import jax.numpy as jnp
from jax.experimental import pallas as pl

TOP_K = 2
N_EXPERTS = 8
D_MODEL = 1024
NSTREAMS = 2
SUB_BLOCK = 1024
TOKENS_PER_BLOCK = NSTREAMS * SUB_BLOCK


def _router_kernel(*refs):
    x_refs = refs[:NSTREAMS]
    w_ref, idx_ref, wgt_ref = refs[NSTREAMS:]
    w = w_ref[...]
    dn = (((1,), (1,)), ((), ()))
    logits = jnp.concatenate(
        [jax.lax.dot_general(w, x_ref[...], dn,
                             preferred_element_type=jnp.float32)
         for x_ref in x_refs], axis=1)

    m = jnp.max(logits, axis=0, keepdims=True)
    u = jnp.exp(logits - m)
    s = u / jnp.sum(u, axis=0, keepdims=True)

    exp_row = jax.lax.broadcasted_iota(jnp.int32, s.shape, 0)
    v1 = jnp.max(s, axis=0, keepdims=True)
    i1 = jnp.min(jnp.where(s == v1, exp_row, N_EXPERTS),
                 axis=0, keepdims=True)
    masked = jnp.where(exp_row == i1, -jnp.inf, s)
    v2 = jnp.max(masked, axis=0, keepdims=True)
    i2 = jnp.min(jnp.where(masked == v2, exp_row, N_EXPERTS),
                 axis=0, keepdims=True)

    denom = (v1 + v2) + 1e-20
    idx_ref[...] = jnp.concatenate([i1, i2], axis=0)
    wgt_ref[...] = jnp.concatenate([v1 / denom, v2 / denom], axis=0)


@jax.jit
def kernel(hidden_states, weight):
    h = hidden_states.shape[-1]
    x = hidden_states.reshape(-1, h).astype(jnp.float32)
    t = x.shape[0]
    w = weight.astype(jnp.float32)
    b = TOKENS_PER_BLOCK
    grid = (t // b,)

    def make_spec(j):
        return pl.BlockSpec((SUB_BLOCK, h), lambda i, j=j: (i * NSTREAMS + j, 0))

    idx_t, wgt_t = pl.pallas_call(
        _router_kernel,
        grid=grid,
        in_specs=[make_spec(j) for j in range(NSTREAMS)] + [
            pl.BlockSpec((N_EXPERTS, h), lambda i: (0, 0)),
        ],
        out_specs=[
            pl.BlockSpec((TOP_K, b), lambda i: (0, i)),
            pl.BlockSpec((TOP_K, b), lambda i: (0, i)),
        ],
        out_shape=[
            jax.ShapeDtypeStruct((TOP_K, t), jnp.int32),
            jax.ShapeDtypeStruct((TOP_K, t), jnp.float32),
        ],
    )(*([x] * NSTREAMS), w)
    return (idx_t.T, wgt_t.T)
